# baseline (speedup 1.0000x reference)
import functools
import math

import jax
import jax.numpy as jnp
from jax.experimental import pallas as pl


M = 4
DIN = 512
DH = 256
DMID = (DIN + DH) // 2


def _branch_kernel(x_ref, att_ref, w1_ref, b1_ref, w2_ref, b2_ref, w3_ref,
                   b3_ref, qw_ref, qb_ref, kw_ref, kb_ref, vw_ref, vb_ref,
                   beta_ref, o_ref):
    f32 = jnp.float32
    zs = []
    for m in range(M):
        xm = x_ref[m] * att_ref[m]
        h = jnp.dot(xm, w1_ref[m], preferred_element_type=f32) + b1_ref[m]
        h = jnp.maximum(h, 0.0)
        h = jnp.dot(h, w2_ref[m], preferred_element_type=f32) + b2_ref[m]
        h = jnp.maximum(h, 0.0)
        z = jnp.dot(h, w3_ref[m], preferred_element_type=f32) + b3_ref[m]
        zs.append(z)

    qw = qw_ref[:]
    kw = kw_ref[:]
    qs = [jnp.dot(z, qw, preferred_element_type=f32) + qb_ref[:] for z in zs]
    ks = [jnp.dot(z, kw, preferred_element_type=f32) + kb_ref[:] for z in zs]

    scale = 1.0 / math.sqrt(float(DH))
    s = [[jnp.sum(qs[i] * ks[j], axis=-1, keepdims=True) * scale
          for j in range(M)] for i in range(M)]

    wj = [0.0] * M
    for i in range(M):
        mx = jnp.maximum(jnp.maximum(s[i][0], s[i][1]),
                         jnp.maximum(s[i][2], s[i][3]))
        e = [jnp.exp(s[i][j] - mx) for j in range(M)]
        inv = 1.0 / (e[0] + e[1] + e[2] + e[3])
        for j in range(M):
            wj[j] = wj[j] + e[j] * inv

    vin = wj[0] * zs[0] + wj[1] * zs[1] + wj[2] * zs[2] + wj[3] * zs[3]
    zmean = (zs[0] + zs[1] + zs[2] + zs[3]) * (1.0 / M)
    beta = beta_ref[0, 0]
    vout = jnp.dot(vin, vw_ref[:], preferred_element_type=f32) * (1.0 / M)
    o_ref[:] = zmean + beta * (vout + vb_ref[:])


def _run_branch(x, att, w1, b1, w2, b2, w3, b3, qw, qb, kw, kb, vw, vb, beta,
                block_n):
    n = x.shape[1]
    npad = ((n + block_n - 1) // block_n) * block_n
    if npad != n:
        x = jnp.pad(x, ((0, 0), (0, npad - n), (0, 0)))
        att = jnp.pad(att, ((0, 0), (0, npad - n), (0, 0)))
    grid = npad // block_n

    full = lambda *shape: pl.BlockSpec(shape, lambda i: (0,) * len(shape))
    out = pl.pallas_call(
        _branch_kernel,
        grid=(grid,),
        in_specs=[
            pl.BlockSpec((M, block_n, DIN), lambda i: (0, i, 0)),
            pl.BlockSpec((M, block_n, 1), lambda i: (0, i, 0)),
            full(M, DIN, DMID), full(M, 1, DMID),
            full(M, DMID, DMID), full(M, 1, DMID),
            full(M, DMID, DH), full(M, 1, DH),
            full(DH, DH), full(1, DH),
            full(DH, DH), full(1, DH),
            full(DH, DH), full(1, DH),
            full(1, 1),
        ],
        out_specs=pl.BlockSpec((block_n, DH), lambda i: (i, 0)),
        out_shape=jax.ShapeDtypeStruct((npad, DH), jnp.float32),
    )(x, att, w1, b1, w2, b2, w3, b3, qw, jnp.reshape(qb, (1, DH)),
      kw, jnp.reshape(kb, (1, DH)), vw, jnp.reshape(vb, (1, DH)),
      jnp.reshape(beta, (1, 1)))
    return out[:n]


@jax.jit
def kernel(drug_fea_tensor, protein_fea_tensor, drug_att, protein_att, dW1,
           db1, dW2, db2, dW3, db3, pW1, pb1, pW2, pb2, pW3, pb3, QdW, Qdb,
           KdW, Kdb, VdW, Vdb, QpW, Qpb, KpW, Kpb, VpW, Vpb, beta_drug,
           beta_protein):
    drug_emb = _run_branch(drug_fea_tensor, drug_att, dW1, db1, dW2, db2,
                           dW3, db3, QdW, Qdb, KdW, Kdb, VdW, Vdb, beta_drug,
                           block_n=128)
    protein_emb = _run_branch(protein_fea_tensor, protein_att, pW1, pb1, pW2,
                              pb2, pW3, pb3, QpW, Qpb, KpW, Kpb, VpW, Vpb,
                              beta_protein, block_n=128)
    return (drug_emb, protein_emb)


# speedup vs baseline: 1.1051x; 1.1051x over previous
import functools
import math

import jax
import jax.numpy as jnp
from jax.experimental import pallas as pl


M = 4
DIN = 512
DH = 256
DMID = (DIN + DH) // 2


def _branch_kernel(x_ref, att_ref, w1_ref, b1_ref, w2_ref, b2_ref, w3_ref,
                   b3_ref, qw_ref, qb_ref, kw_ref, kb_ref, vw_ref, vb_ref,
                   beta_ref, o_ref):
    f32 = jnp.float32
    bf16 = jnp.bfloat16
    zs = []
    z16s = []
    for m in range(M):
        xm = x_ref[m] * att_ref[m]
        h = jnp.dot(xm, w1_ref[m], preferred_element_type=f32) + b1_ref[m]
        h = jnp.maximum(h, 0.0).astype(bf16)
        h = jnp.dot(h, w2_ref[m], preferred_element_type=f32) + b2_ref[m]
        h = jnp.maximum(h, 0.0).astype(bf16)
        z = jnp.dot(h, w3_ref[m], preferred_element_type=f32) + b3_ref[m]
        zs.append(z)
        z16s.append(z.astype(bf16))

    qw = qw_ref[:]
    kw = kw_ref[:]
    qs = [jnp.dot(z, qw, preferred_element_type=f32) + qb_ref[:] for z in z16s]
    ks = [jnp.dot(z, kw, preferred_element_type=f32) + kb_ref[:] for z in z16s]

    scale = 1.0 / math.sqrt(float(DH))
    s = [[jnp.sum(qs[i] * ks[j], axis=-1, keepdims=True) * scale
          for j in range(M)] for i in range(M)]

    wj = [0.0] * M
    for i in range(M):
        mx = jnp.maximum(jnp.maximum(s[i][0], s[i][1]),
                         jnp.maximum(s[i][2], s[i][3]))
        e = [jnp.exp(s[i][j] - mx) for j in range(M)]
        inv = 1.0 / (e[0] + e[1] + e[2] + e[3])
        for j in range(M):
            wj[j] = wj[j] + e[j] * inv

    vin = wj[0] * zs[0] + wj[1] * zs[1] + wj[2] * zs[2] + wj[3] * zs[3]
    zmean = (zs[0] + zs[1] + zs[2] + zs[3]) * (1.0 / M)
    beta = beta_ref[0, 0]
    vout = jnp.dot(vin.astype(bf16), vw_ref[:],
                   preferred_element_type=f32) * (1.0 / M)
    o_ref[:] = zmean + beta * (vout + vb_ref[:])


def _run_branch(x, att, w1, b1, w2, b2, w3, b3, qw, qb, kw, kb, vw, vb, beta,
                block_n):
    n = x.shape[1]
    npad = ((n + block_n - 1) // block_n) * block_n
    x = x.astype(jnp.bfloat16)
    att = att.astype(jnp.bfloat16)
    if npad != n:
        x = jnp.pad(x, ((0, 0), (0, npad - n), (0, 0)))
        att = jnp.pad(att, ((0, 0), (0, npad - n), (0, 0)))
    w1, w2, w3, qw, kw, vw = (a.astype(jnp.bfloat16)
                              for a in (w1, w2, w3, qw, kw, vw))
    grid = npad // block_n

    full = lambda *shape: pl.BlockSpec(shape, lambda i: (0,) * len(shape))
    out = pl.pallas_call(
        _branch_kernel,
        grid=(grid,),
        in_specs=[
            pl.BlockSpec((M, block_n, DIN), lambda i: (0, i, 0)),
            pl.BlockSpec((M, block_n, 1), lambda i: (0, i, 0)),
            full(M, DIN, DMID), full(M, 1, DMID),
            full(M, DMID, DMID), full(M, 1, DMID),
            full(M, DMID, DH), full(M, 1, DH),
            full(DH, DH), full(1, DH),
            full(DH, DH), full(1, DH),
            full(DH, DH), full(1, DH),
            full(1, 1),
        ],
        out_specs=pl.BlockSpec((block_n, DH), lambda i: (i, 0)),
        out_shape=jax.ShapeDtypeStruct((npad, DH), jnp.float32),
    )(x, att, w1, b1, w2, b2, w3, b3, qw, jnp.reshape(qb, (1, DH)),
      kw, jnp.reshape(kb, (1, DH)), vw, jnp.reshape(vb, (1, DH)),
      jnp.reshape(beta, (1, 1)))
    return out[:n]


@jax.jit
def kernel(drug_fea_tensor, protein_fea_tensor, drug_att, protein_att, dW1,
           db1, dW2, db2, dW3, db3, pW1, pb1, pW2, pb2, pW3, pb3, QdW, Qdb,
           KdW, Kdb, VdW, Vdb, QpW, Qpb, KpW, Kpb, VpW, Vpb, beta_drug,
           beta_protein):
    drug_emb = _run_branch(drug_fea_tensor, drug_att, dW1, db1, dW2, db2,
                           dW3, db3, QdW, Qdb, KdW, Kdb, VdW, Vdb, beta_drug,
                           block_n=128)
    protein_emb = _run_branch(protein_fea_tensor, protein_att, pW1, pb1, pW2,
                              pb2, pW3, pb3, QpW, Qpb, KpW, Kpb, VpW, Vpb,
                              beta_protein, block_n=128)
    return (drug_emb, protein_emb)


# speedup vs baseline: 1.3177x; 1.1924x over previous
import math

import jax
import jax.numpy as jnp
from jax.experimental import pallas as pl


M = 4
ND = 708
NP = 1512
DIN = 512
DH = 256
DMID = (DIN + DH) // 2

BN = 256
DBLK = (ND + BN - 1) // BN
PBLK = (NP + BN - 1) // BN
NROWS = (DBLK + PBLK) * BN


def _branch_body(x_ref, att_ref, w1_ref, b1_ref, w2_ref, b2_ref, w3_ref,
                 b3_ref, qkw_ref, qb_ref, kb_ref, vw_ref, vb_ref, beta_ref,
                 o_ref):
    f32 = jnp.float32
    bf16 = jnp.bfloat16
    zs = []
    qks = []
    for m in range(M):
        xm = (x_ref[m] * att_ref[m]).astype(bf16)
        h = jnp.dot(xm, w1_ref[m], preferred_element_type=f32) + b1_ref[m]
        h = jnp.maximum(h, 0.0).astype(bf16)
        h = jnp.dot(h, w2_ref[m], preferred_element_type=f32) + b2_ref[m]
        h = jnp.maximum(h, 0.0).astype(bf16)
        z = jnp.dot(h, w3_ref[m], preferred_element_type=f32) + b3_ref[m]
        zs.append(z)
        qks.append(jnp.dot(z.astype(bf16), qkw_ref[:],
                           preferred_element_type=f32))

    qs = [qk[:, :DH] + qb_ref[:] for qk in qks]
    ks = [qk[:, DH:] + kb_ref[:] for qk in qks]

    scale = 1.0 / math.sqrt(float(DH))
    s = [[jnp.sum(qs[i] * ks[j], axis=-1, keepdims=True) * scale
          for j in range(M)] for i in range(M)]

    wj = [0.0] * M
    for i in range(M):
        mx = jnp.maximum(jnp.maximum(s[i][0], s[i][1]),
                         jnp.maximum(s[i][2], s[i][3]))
        e = [jnp.exp(s[i][j] - mx) for j in range(M)]
        inv = 1.0 / (e[0] + e[1] + e[2] + e[3])
        for j in range(M):
            wj[j] = wj[j] + e[j] * inv

    vin = wj[0] * zs[0] + wj[1] * zs[1] + wj[2] * zs[2] + wj[3] * zs[3]
    zmean = (zs[0] + zs[1] + zs[2] + zs[3]) * (1.0 / M)
    beta = beta_ref[0, 0]
    vout = jnp.dot(vin.astype(bf16), vw_ref[:],
                   preferred_element_type=f32) * (1.0 / M)
    o_ref[:] = zmean + beta * (vout + vb_ref[:])


def _kernel_body(xd_ref, attd_ref, xp_ref, attp_ref,
                 dw1_ref, db1_ref, dw2_ref, db2_ref, dw3_ref, db3_ref,
                 dqkw_ref, dqb_ref, dkb_ref, dvw_ref, dvb_ref, dbeta_ref,
                 pw1_ref, pb1_ref, pw2_ref, pb2_ref, pw3_ref, pb3_ref,
                 pqkw_ref, pqb_ref, pkb_ref, pvw_ref, pvb_ref, pbeta_ref,
                 o_ref):
    i = pl.program_id(0)

    @pl.when(i < DBLK)
    def _():
        _branch_body(xd_ref, attd_ref, dw1_ref, db1_ref, dw2_ref, db2_ref,
                     dw3_ref, db3_ref, dqkw_ref, dqb_ref, dkb_ref, dvw_ref,
                     dvb_ref, dbeta_ref, o_ref)

    @pl.when(i >= DBLK)
    def _():
        _branch_body(xp_ref, attp_ref, pw1_ref, pb1_ref, pw2_ref, pb2_ref,
                     pw3_ref, pb3_ref, pqkw_ref, pqb_ref, pkb_ref, pvw_ref,
                     pvb_ref, pbeta_ref, o_ref)


def _full(*shape):
    return pl.BlockSpec(shape, lambda i: (0,) * len(shape))


@jax.jit
def kernel(drug_fea_tensor, protein_fea_tensor, drug_att, protein_att, dW1,
           db1, dW2, db2, dW3, db3, pW1, pb1, pW2, pb2, pW3, pb3, QdW, Qdb,
           KdW, Kdb, VdW, Vdb, QpW, Qpb, KpW, Kpb, VpW, Vpb, beta_drug,
           beta_protein):
    bf16 = jnp.bfloat16
    dqkw = jnp.concatenate([QdW, KdW], axis=1).astype(bf16)
    pqkw = jnp.concatenate([QpW, KpW], axis=1).astype(bf16)
    dws = [w.astype(bf16) for w in (dW1, dW2, dW3)]
    pws = [w.astype(bf16) for w in (pW1, pW2, pW3)]

    out = pl.pallas_call(
        _kernel_body,
        grid=(DBLK + PBLK,),
        in_specs=[
            pl.BlockSpec((M, BN, DIN), lambda i: (0, jnp.minimum(i, DBLK - 1), 0)),
            pl.BlockSpec((M, BN, 1), lambda i: (0, jnp.minimum(i, DBLK - 1), 0)),
            pl.BlockSpec((M, BN, DIN), lambda i: (0, jnp.maximum(i - DBLK, 0), 0)),
            pl.BlockSpec((M, BN, 1), lambda i: (0, jnp.maximum(i - DBLK, 0), 0)),
            _full(M, DIN, DMID), _full(M, 1, DMID),
            _full(M, DMID, DMID), _full(M, 1, DMID),
            _full(M, DMID, DH), _full(M, 1, DH),
            _full(DH, 2 * DH), _full(1, DH), _full(1, DH),
            _full(DH, DH), _full(1, DH), _full(1, 1),
            _full(M, DIN, DMID), _full(M, 1, DMID),
            _full(M, DMID, DMID), _full(M, 1, DMID),
            _full(M, DMID, DH), _full(M, 1, DH),
            _full(DH, 2 * DH), _full(1, DH), _full(1, DH),
            _full(DH, DH), _full(1, DH), _full(1, 1),
        ],
        out_specs=pl.BlockSpec((BN, DH), lambda i: (i, 0)),
        out_shape=jax.ShapeDtypeStruct((NROWS, DH), jnp.float32),
    )(drug_fea_tensor, drug_att, protein_fea_tensor, protein_att,
      dws[0], db1, dws[1], db2, dws[2], db3,
      dqkw, jnp.reshape(Qdb, (1, DH)), jnp.reshape(Kdb, (1, DH)),
      VdW.astype(bf16), jnp.reshape(Vdb, (1, DH)),
      jnp.reshape(beta_drug, (1, 1)),
      pws[0], pb1, pws[1], pb2, pws[2], pb3,
      pqkw, jnp.reshape(Qpb, (1, DH)), jnp.reshape(Kpb, (1, DH)),
      VpW.astype(bf16), jnp.reshape(Vpb, (1, DH)),
      jnp.reshape(beta_protein, (1, 1)))

    drug_emb = out[:ND]
    protein_emb = out[DBLK * BN:DBLK * BN + NP]
    return (drug_emb, protein_emb)


# speedup vs baseline: 1.5375x; 1.1668x over previous
import math

import jax
import jax.numpy as jnp
from jax.experimental import pallas as pl
from jax.experimental.pallas import tpu as pltpu


M = 4
ND = 708
NP = 1512
DIN = 512
DH = 256
DMID = (DIN + DH) // 2

BN = 256
DBLK = (ND + BN - 1) // BN
PBLK = (NP + BN - 1) // BN
NROWS = (DBLK + PBLK) * BN

F32 = jnp.float32
BF16 = jnp.bfloat16


def _branch_body(x_ref, att_ref, w1_ref, w2_ref, w3_ref, qkw_ref, vw_ref,
                 b1_ref, b2_ref, b3_ref, qb_ref, kb_ref, vb_ref, beta_ref,
                 o_ref):
    z16s = []
    qks = []
    zsum = None
    for m in range(M):
        xm = (x_ref[m] * att_ref[m]).astype(BF16)
        h = jnp.dot(xm, w1_ref[m], preferred_element_type=F32)
        h = jnp.maximum(h.astype(BF16) + b1_ref[m].astype(BF16), 0.0)
        h = jnp.dot(h, w2_ref[m], preferred_element_type=F32)
        h = jnp.maximum(h.astype(BF16) + b2_ref[m].astype(BF16), 0.0)
        z = jnp.dot(h, w3_ref[m], preferred_element_type=F32) + b3_ref[m]
        zsum = z if zsum is None else zsum + z
        z16 = z.astype(BF16)
        z16s.append(z16)
        qks.append(jnp.dot(z16, qkw_ref[:], preferred_element_type=F32))

    qs = [qk[:, :DH] + qb_ref[:] for qk in qks]
    ks = [qk[:, DH:] + kb_ref[:] for qk in qks]

    scale = 1.0 / math.sqrt(float(DH))
    s = [[jnp.sum(qs[i] * ks[j], axis=-1, keepdims=True) * scale
          for j in range(M)] for i in range(M)]

    wj = [0.0] * M
    for i in range(M):
        mx = jnp.maximum(jnp.maximum(s[i][0], s[i][1]),
                         jnp.maximum(s[i][2], s[i][3]))
        e = [jnp.exp(s[i][j] - mx) for j in range(M)]
        inv = 1.0 / (e[0] + e[1] + e[2] + e[3])
        for j in range(M):
            wj[j] = wj[j] + e[j] * inv

    vin = sum(wj[j].astype(BF16) * z16s[j] for j in range(M))
    beta = beta_ref[0, 0]
    vout = jnp.dot(vin, vw_ref[:], preferred_element_type=F32) * (1.0 / M)
    o_ref[:] = zsum * (1.0 / M) + beta * (vout + vb_ref[:])


def _kernel_body(xd_ref, attd_ref, xp_ref, attp_ref,
                 dw1_ref, dw2_ref, dw3_ref, dqw_ref, dkw_ref, dvw_ref,
                 db1_ref, db2_ref, db3_ref, dqb_ref, dkb_ref, dvb_ref,
                 pw1_ref, pw2_ref, pw3_ref, pqw_ref, pkw_ref, pvw_ref,
                 pb1_ref, pb2_ref, pb3_ref, pqb_ref, pkb_ref, pvb_ref,
                 dbeta_ref, pbeta_ref,
                 o_ref,
                 dw1s, dw2s, dw3s, dqkws, dvws,
                 pw1s, pw2s, pw3s, pqkws, pvws):
    i = pl.program_id(0)

    @pl.when(i == 0)
    def _():
        dw1s[:] = dw1_ref[:].astype(BF16)
        dw2s[:] = dw2_ref[:].astype(BF16)
        dw3s[:] = dw3_ref[:].astype(BF16)
        dqkws[:, :DH] = dqw_ref[:].astype(BF16)
        dqkws[:, DH:] = dkw_ref[:].astype(BF16)
        dvws[:] = dvw_ref[:].astype(BF16)
        pw1s[:] = pw1_ref[:].astype(BF16)
        pw2s[:] = pw2_ref[:].astype(BF16)
        pw3s[:] = pw3_ref[:].astype(BF16)
        pqkws[:, :DH] = pqw_ref[:].astype(BF16)
        pqkws[:, DH:] = pkw_ref[:].astype(BF16)
        pvws[:] = pvw_ref[:].astype(BF16)

    @pl.when(i < DBLK)
    def _():
        _branch_body(xd_ref, attd_ref, dw1s, dw2s, dw3s, dqkws, dvws,
                     db1_ref, db2_ref, db3_ref, dqb_ref, dkb_ref, dvb_ref,
                     dbeta_ref, o_ref)

    @pl.when(i >= DBLK)
    def _():
        _branch_body(xp_ref, attp_ref, pw1s, pw2s, pw3s, pqkws, pvws,
                     pb1_ref, pb2_ref, pb3_ref, pqb_ref, pkb_ref, pvb_ref,
                     pbeta_ref, o_ref)


_VMEM = pl.BlockSpec(memory_space=pltpu.VMEM)
_SMEM = pl.BlockSpec(memory_space=pltpu.SMEM)


@jax.jit
def kernel(drug_fea_tensor, protein_fea_tensor, drug_att, protein_att, dW1,
           db1, dW2, db2, dW3, db3, pW1, pb1, pW2, pb2, pW3, pb3, QdW, Qdb,
           KdW, Kdb, VdW, Vdb, QpW, Qpb, KpW, Kpb, VpW, Vpb, beta_drug,
           beta_protein):
    out = pl.pallas_call(
        _kernel_body,
        grid=(DBLK + PBLK,),
        in_specs=(
            [pl.BlockSpec((M, BN, DIN), lambda i: (0, jnp.minimum(i, DBLK - 1), 0)),
             pl.BlockSpec((M, BN, 1), lambda i: (0, jnp.minimum(i, DBLK - 1), 0)),
             pl.BlockSpec((M, BN, DIN), lambda i: (0, jnp.maximum(i - DBLK, 0), 0)),
             pl.BlockSpec((M, BN, 1), lambda i: (0, jnp.maximum(i - DBLK, 0), 0))]
            + [_VMEM] * 24 + [_SMEM] * 2
        ),
        out_specs=pl.BlockSpec((BN, DH), lambda i: (i, 0)),
        out_shape=jax.ShapeDtypeStruct((NROWS, DH), F32),
        scratch_shapes=[
            pltpu.VMEM((M, DIN, DMID), BF16), pltpu.VMEM((M, DMID, DMID), BF16),
            pltpu.VMEM((M, DMID, DH), BF16), pltpu.VMEM((DH, 2 * DH), BF16),
            pltpu.VMEM((DH, DH), BF16),
            pltpu.VMEM((M, DIN, DMID), BF16), pltpu.VMEM((M, DMID, DMID), BF16),
            pltpu.VMEM((M, DMID, DH), BF16), pltpu.VMEM((DH, 2 * DH), BF16),
            pltpu.VMEM((DH, DH), BF16),
        ],
    )(drug_fea_tensor, drug_att, protein_fea_tensor, protein_att,
      dW1, dW2, dW3, QdW, KdW, VdW,
      db1, db2, db3, jnp.reshape(Qdb, (1, DH)), jnp.reshape(Kdb, (1, DH)),
      jnp.reshape(Vdb, (1, DH)),
      pW1, pW2, pW3, QpW, KpW, VpW,
      pb1, pb2, pb3, jnp.reshape(Qpb, (1, DH)), jnp.reshape(Kpb, (1, DH)),
      jnp.reshape(Vpb, (1, DH)),
      jnp.reshape(beta_drug, (1, 1)), jnp.reshape(beta_protein, (1, 1)))

    drug_emb = out[:ND]
    protein_emb = out[DBLK * BN:DBLK * BN + NP]
    return (drug_emb, protein_emb)
